# baseline (speedup 1.0000x reference)
import functools
import math

import jax
import jax.numpy as jnp
import numpy as np
from jax import lax
from jax.experimental import pallas as pl
from jax.experimental.pallas import tpu as pltpu
from jax.experimental.pallas import tpu_sc as plsc

N = 10000
E = 320000
D = 128
HID = 64
K = 10

NT = 16
NP = 10240
STRIPE = NP // NT
CH = 128
NCHUNK = 160
NCHUNK_D = 79
DUMMY = N
BLK = 512

_B = np.zeros((16, 16), np.float64)
for _m in range(K + 1):
    _p = np.array([1.0])
    for _ in range(_m):
        _p = np.convolve(_p, [1.0, -1.0])
    for _ in range(K - _m):
        _p = np.convolve(_p, [1.0, 1.0])
    _B[_m, : len(_p)] = _p
_BMAT = np.asarray(_B, np.float32)
_CWn = np.zeros((1, 16), np.float64)
_CWn[0, : K + 1] = [math.comb(K, m) / 2.0 ** K for m in range(K + 1)]
_CW = np.asarray(_CWn, np.float32)

_MESH = plsc.VectorSubcoreMesh(core_axis_name="c", subcore_axis_name="s")
_SC_PARAMS = pltpu.CompilerParams(use_tc_tiling_on_sc=False)


def _deg_body(srcd, degp, sdeg_sh, idx_v, ones_v, zero_v):
    cid = lax.axis_index("c")
    sid = lax.axis_index("s")
    nbase = sid * STRIPE
    nsl = pl.ds(nbase, STRIPE)

    def _fill(r, _):
        ones_v[r, :] = jnp.full((16,), 1.0, jnp.float32)
        zero_v[r, :] = jnp.zeros((16,), jnp.float32)
        return 0

    lax.fori_loop(0, CH, _fill, 0)
    for q in range(STRIPE // CH):
        pltpu.sync_copy(zero_v, sdeg_sh.at[pl.ds(nbase + q * CH, CH)])
    pltpu.sync_copy(srcd.at[cid, sid], idx_v)
    plsc.subcore_barrier()

    def _chunk(i, _):
        pltpu.sync_copy(ones_v, sdeg_sh.at[idx_v.at[i]], add=True)
        return 0

    lax.fori_loop(0, NCHUNK_D, _chunk, 0)
    plsc.subcore_barrier()
    pltpu.sync_copy(sdeg_sh.at[nsl], degp.at[cid, nsl])


_deg_call = functools.partial(
    pl.kernel,
    out_type=jax.ShapeDtypeStruct((2, NP, 16), jnp.float32),
    mesh=_MESH,
    compiler_params=_SC_PARAMS,
    scratch_types=[
        pltpu.VMEM_SHARED((NP, 16), jnp.float32),
        pltpu.VMEM((NCHUNK_D, CH), jnp.int32),
        pltpu.VMEM((CH, 16), jnp.float32),
        pltpu.VMEM((CH, 16), jnp.float32),
    ],
)(_deg_body)


def _tc_body(temp_ref, cw_ref, bmat_ref, x_ref, w1_ref, b1_ref, w2_ref,
             b2_ref, degp_ref, w0_ref, acc0_ref, d2x_ref, adx_ref):
    h1 = jnp.maximum(x_ref[...] @ w1_ref[...] + b1_ref[...], 0.0)
    h = h1 @ w2_ref[...] + b2_ref[...]
    deg = degp_ref[0, :, 0:1] + degp_ref[1, :, 0:1]
    dinv = jnp.where(deg > 0, lax.rsqrt(deg), 0.0)
    tvec = jnp.maximum(temp_ref[...], 0.0)
    avec = (tvec * cw_ref[...]) @ bmat_ref[...]
    hw = h * dinv
    ha = h * avec[0:1, 0:1]
    w0_ref[...] = jnp.stack([hw[:, :32], hw[:, 32:]], axis=0)
    acc0_ref[...] = jnp.stack([ha[:, :32], ha[:, 32:]], axis=0)
    d2x_ref[...] = jnp.broadcast_to(dinv * dinv, (BLK, 16))
    ad = avec[0, 1 : K + 1]
    adx_ref[...] = jnp.broadcast_to(
        ad[:, None, None] * dinv[None, :, :], (K, BLK, 16))


def _tc_call(temp2, xpad, W1, b1r, W2, b2r, degp):
    full = lambda s: pl.BlockSpec(s, lambda i: (0,) * len(s))
    return pl.pallas_call(
        _tc_body,
        grid=(NP // BLK,),
        in_specs=[
            full((1, 16)),
            full((1, 16)),
            full((16, 16)),
            pl.BlockSpec((BLK, D), lambda i: (i, 0)),
            full((D, HID)),
            full((1, HID)),
            full((HID, HID)),
            full((1, HID)),
            pl.BlockSpec((2, BLK, 16), lambda i: (0, i, 0)),
        ],
        out_specs=[
            pl.BlockSpec((2, BLK, 32), lambda i: (0, i, 0)),
            pl.BlockSpec((2, BLK, 32), lambda i: (0, i, 0)),
            pl.BlockSpec((BLK, 16), lambda i: (i, 0)),
            pl.BlockSpec((K, BLK, 16), lambda i: (0, i, 0)),
        ],
        out_shape=[
            jax.ShapeDtypeStruct((2, NP, 32), jnp.float32),
            jax.ShapeDtypeStruct((2, NP, 32), jnp.float32),
            jax.ShapeDtypeStruct((NP, 16), jnp.float32),
            jax.ShapeDtypeStruct((K, NP, 16), jnp.float32),
        ],
    )(temp2, jnp.asarray(_CW), jnp.asarray(_BMAT), xpad, W1, b1r, W2, b2r,
      degp)


G = 16
NG = NCHUNK // G
NQ = STRIPE // CH
RING = 4
AHEAD = RING // 2


def _prop_body(w0t, acc0t, d2x, adx, srcp, dstp, accout,
               w_sh, s_sh, acc_v, srcb, dstb, sbuf2, d2xb2, advb2,
               gbuf, zbuf, gsems, ssems, isems, rsems, wsem, zsem):
    cid = lax.axis_index("c")
    sid = lax.axis_index("s")
    nbase = sid * STRIPE
    nsl = pl.ds(nbase, STRIPE)

    def _drain16(sem):
        pltpu.make_async_copy(w0t.at[0].at[pl.ds(0, CH)], gbuf.at[0],
                              sem).wait()

    def _drain8(sem):
        pltpu.make_async_copy(d2x.at[pl.ds(0, CH)], d2xb2.at[0], sem).wait()

    def _drain8i(sem):
        pltpu.make_async_copy(srcp.at[0, pl.ds(0, G)], srcb.at[0], sem).wait()

    pltpu.sync_copy(w0t.at[cid, nsl], w_sh.at[nsl])
    pltpu.sync_copy(acc0t.at[cid, nsl], acc_v)

    def _zfill(r, _):
        zbuf[r, pl.ds(0, 16)] = jnp.zeros((16,), jnp.float32)
        zbuf[r, pl.ds(16, 16)] = jnp.zeros((16,), jnp.float32)
        return 0

    lax.fori_loop(0, CH, _zfill, 0)
    for q in range(NQ):
        pltpu.sync_copy(zbuf, s_sh.at[pl.ds(nbase + q * CH, CH)])
    plsc.subcore_barrier()

    def _gpair(p, _):
        for gg in range(2):
            g = 2 * p + gg
            _group_body(g, gg, 1 - gg)
        return 0

    def _group_body(g, slot, nslot):
        for b in range(G):
            c = g * G + b
            rb = b % RING

            @pl.when(c >= AHEAD)
            def _():
                _drain16(ssems.at[(rb + AHEAD) % RING])

            if b == 4:
                @pl.when(g + 1 < NG)
                def _():
                    pltpu.async_copy(srcp.at[sid, pl.ds((g + 1) * G, G)],
                                     srcb.at[nslot], isems.at[0])
                    pltpu.async_copy(dstp.at[sid, pl.ds((g + 1) * G, G)],
                                     dstb.at[nslot], isems.at[1])

            if b == G - AHEAD:
                @pl.when(g + 1 < NG)
                def _():
                    _drain8i(isems.at[0])
                    _drain8i(isems.at[1])

            if b < G - AHEAD:
                srow = srcb.at[slot, b + AHEAD]
            else:
                srow = srcb.at[nslot, b - (G - AHEAD)]

            @pl.when(c + AHEAD < NCHUNK)
            def _():
                pltpu.async_copy(w_sh.at[srow], gbuf.at[(rb + AHEAD) % RING],
                                 gsems.at[(rb + AHEAD) % RING])

            _drain16(gsems.at[rb])
            pltpu.async_copy(gbuf.at[rb], s_sh.at[dstb.at[slot, b]],
                             ssems.at[rb], add=True)

    def _rload(j, q, slot):
        qsl = pl.ds(nbase + q * CH, CH)
        pltpu.async_copy(s_sh.at[qsl], sbuf2.at[slot], rsems.at[slot])
        pltpu.async_copy(d2x.at[qsl], d2xb2.at[slot], rsems.at[slot])
        pltpu.async_copy(adx.at[j].at[qsl], advb2.at[slot], rsems.at[slot])

    def _rescale(j):
        for q in range(NQ):
            slot = 0
            qsl0 = pl.ds(nbase + q * CH, CH)
            pltpu.sync_copy(s_sh.at[qsl0], sbuf2.at[slot])
            pltpu.sync_copy(d2x.at[qsl0], d2xb2.at[slot])
            pltpu.sync_copy(adx.at[j].at[qsl0], advb2.at[slot])

            def _row(r, _):
                s0 = sbuf2[slot, r, pl.ds(0, 16)]
                s1 = sbuf2[slot, r, pl.ds(16, 16)]
                ad = advb2[slot, r, :]
                d2 = d2xb2[slot, r, :]
                ar = q * CH + r
                acc_v[ar, pl.ds(0, 16)] = acc_v[ar, pl.ds(0, 16)] + ad * s0
                acc_v[ar, pl.ds(16, 16)] = acc_v[ar, pl.ds(16, 16)] + ad * s1
                sbuf2[slot, r, pl.ds(0, 16)] = d2 * s0
                sbuf2[slot, r, pl.ds(16, 16)] = d2 * s1
                return 0

            lax.fori_loop(0, CH, _row, 0)
            qsl = pl.ds(nbase + q * CH, CH)
            pltpu.sync_copy(sbuf2.at[slot], w_sh.at[qsl])
            pltpu.sync_copy(zbuf, s_sh.at[qsl])

    def _step(j, _):
        pltpu.async_copy(srcp.at[sid, pl.ds(0, G)], srcb.at[0], isems.at[0])
        pltpu.async_copy(dstp.at[sid, pl.ds(0, G)], dstb.at[0], isems.at[1])
        _drain8i(isems.at[0])
        _drain8i(isems.at[1])
        for b in range(AHEAD):
            pltpu.async_copy(w_sh.at[srcb.at[0, b]], gbuf.at[b], gsems.at[b])
        lax.fori_loop(0, NG // 2, _gpair, 0)
        for rb in range(RING - AHEAD, RING):
            _drain16(ssems.at[rb])
        plsc.subcore_barrier()
        _rescale(j)
        plsc.subcore_barrier()
        return 0

    lax.fori_loop(0, K, _step, 0)
    pltpu.sync_copy(acc_v, accout.at[cid, nsl])


_prop_call = functools.partial(
    pl.kernel,
    out_type=jax.ShapeDtypeStruct((2, NP, 32), jnp.float32),
    mesh=_MESH,
    compiler_params=_SC_PARAMS,
    scratch_types=[
        pltpu.VMEM_SHARED((NP, 32), jnp.float32),
        pltpu.VMEM_SHARED((NP, 32), jnp.float32),
        pltpu.VMEM((STRIPE, 32), jnp.float32),
        pltpu.VMEM((2, G, CH), jnp.int32),
        pltpu.VMEM((2, G, CH), jnp.int32),
        pltpu.VMEM((2, CH, 32), jnp.float32),
        pltpu.VMEM((2, CH, 16), jnp.float32),
        pltpu.VMEM((2, CH, 16), jnp.float32),
        pltpu.VMEM((RING, CH, 32), jnp.float32),
        pltpu.VMEM((CH, 32), jnp.float32),
        pltpu.SemaphoreType.DMA((RING,)),
        pltpu.SemaphoreType.DMA((RING,)),
        pltpu.SemaphoreType.DMA((2,)),
        pltpu.SemaphoreType.DMA((2,)),
        pltpu.SemaphoreType.DMA,
        pltpu.SemaphoreType.DMA,
    ],
)(_prop_body)


def kernel(x, edge_index, epoch, W1, b1, W2, b2, temp):
    src = edge_index[0]
    dst = edge_index[1]
    pad = 2 * NT * NCHUNK_D * CH - E
    srcd = jnp.concatenate(
        [src, jnp.full((pad,), DUMMY, jnp.int32)]).reshape(2, NT, NCHUNK_D, CH)
    degp = _deg_call(srcd)

    temp2 = jnp.pad(temp, (0, 16 - (K + 1))).reshape(1, 16)
    xpad = jnp.pad(x, ((0, NP - N), (0, 0)))
    w0t, acc0t, d2x, adx = _tc_call(
        temp2, xpad, W1, b1.reshape(1, HID), W2, b2.reshape(1, HID), degp)

    padp = NT * NCHUNK * CH - E
    srcp = jnp.concatenate(
        [src, jnp.zeros((padp,), jnp.int32)]).reshape(NT, NCHUNK, CH)
    dstp = jnp.concatenate(
        [dst, jnp.full((padp,), DUMMY, jnp.int32)]).reshape(NT, NCHUNK, CH)

    accout = _prop_call(w0t, acc0t, d2x, adx, srcp, dstp)
    return accout.transpose(1, 0, 2).reshape(NP, HID)[:N]


# speedup vs baseline: 1.0719x; 1.0719x over previous
import functools
import math

import jax
import jax.numpy as jnp
import numpy as np
from jax import lax
from jax.experimental import pallas as pl
from jax.experimental.pallas import tpu as pltpu
from jax.experimental.pallas import tpu_sc as plsc

N = 10000
E = 320000
D = 128
HID = 64
K = 10

NT = 16
NP = 10240
STRIPE = NP // NT
CH = 128
NCHUNK = 160
NCHUNK_D = 79
DUMMY = N
BLK = 512

_B = np.zeros((16, 16), np.float64)
for _m in range(K + 1):
    _p = np.array([1.0])
    for _ in range(_m):
        _p = np.convolve(_p, [1.0, -1.0])
    for _ in range(K - _m):
        _p = np.convolve(_p, [1.0, 1.0])
    _B[_m, : len(_p)] = _p
_BMAT = np.asarray(_B, np.float32)
_CWn = np.zeros((1, 16), np.float64)
_CWn[0, : K + 1] = [math.comb(K, m) / 2.0 ** K for m in range(K + 1)]
_CW = np.asarray(_CWn, np.float32)

_MESH = plsc.VectorSubcoreMesh(core_axis_name="c", subcore_axis_name="s")
_SC_PARAMS = pltpu.CompilerParams(use_tc_tiling_on_sc=False)


def _deg_body(srcd, degp, sdeg_sh, idx_v, ones_v, zero_v):
    cid = lax.axis_index("c")
    sid = lax.axis_index("s")
    nbase = sid * STRIPE
    nsl = pl.ds(nbase, STRIPE)

    def _fill(r, _):
        ones_v[r, :] = jnp.full((16,), 1.0, jnp.float32)
        zero_v[r, :] = jnp.zeros((16,), jnp.float32)
        return 0

    lax.fori_loop(0, CH, _fill, 0)
    for q in range(STRIPE // CH):
        pltpu.sync_copy(zero_v, sdeg_sh.at[pl.ds(nbase + q * CH, CH)])
    pltpu.sync_copy(srcd.at[cid, sid], idx_v)
    plsc.subcore_barrier()

    def _chunk(i, _):
        pltpu.sync_copy(ones_v, sdeg_sh.at[idx_v.at[i]], add=True)
        return 0

    lax.fori_loop(0, NCHUNK_D, _chunk, 0)
    plsc.subcore_barrier()
    pltpu.sync_copy(sdeg_sh.at[nsl], degp.at[cid, nsl])


_deg_call = functools.partial(
    pl.kernel,
    out_type=jax.ShapeDtypeStruct((2, NP, 16), jnp.float32),
    mesh=_MESH,
    compiler_params=_SC_PARAMS,
    scratch_types=[
        pltpu.VMEM_SHARED((NP, 16), jnp.float32),
        pltpu.VMEM((NCHUNK_D, CH), jnp.int32),
        pltpu.VMEM((CH, 16), jnp.float32),
        pltpu.VMEM((CH, 16), jnp.float32),
    ],
)(_deg_body)


def _tc_body(temp_ref, cw_ref, bmat_ref, x_ref, w1_ref, b1_ref, w2_ref,
             b2_ref, degp_ref, w0_ref, acc0_ref, d2x_ref, adx_ref):
    h1 = jnp.maximum(x_ref[...] @ w1_ref[...] + b1_ref[...], 0.0)
    h = h1 @ w2_ref[...] + b2_ref[...]
    deg = degp_ref[0, :, 0:1] + degp_ref[1, :, 0:1]
    dinv = jnp.where(deg > 0, lax.rsqrt(deg), 0.0)
    tvec = jnp.maximum(temp_ref[...], 0.0)
    avec = (tvec * cw_ref[...]) @ bmat_ref[...]
    hw = h * dinv
    ha = h * avec[0:1, 0:1]
    w0_ref[...] = jnp.stack([hw[:, :32], hw[:, 32:]], axis=0)
    acc0_ref[...] = jnp.stack([ha[:, :32], ha[:, 32:]], axis=0)
    d2x_ref[...] = jnp.broadcast_to(dinv * dinv, (BLK, 16))
    ad = avec[0, 1 : K + 1]
    adx_ref[...] = jnp.broadcast_to(
        ad[:, None, None] * dinv[None, :, :], (K, BLK, 16))


def _tc_call(temp2, xpad, W1, b1r, W2, b2r, degp):
    full = lambda s: pl.BlockSpec(s, lambda i: (0,) * len(s))
    return pl.pallas_call(
        _tc_body,
        grid=(NP // BLK,),
        in_specs=[
            full((1, 16)),
            full((1, 16)),
            full((16, 16)),
            pl.BlockSpec((BLK, D), lambda i: (i, 0)),
            full((D, HID)),
            full((1, HID)),
            full((HID, HID)),
            full((1, HID)),
            pl.BlockSpec((2, BLK, 16), lambda i: (0, i, 0)),
        ],
        out_specs=[
            pl.BlockSpec((2, BLK, 32), lambda i: (0, i, 0)),
            pl.BlockSpec((2, BLK, 32), lambda i: (0, i, 0)),
            pl.BlockSpec((BLK, 16), lambda i: (i, 0)),
            pl.BlockSpec((K, BLK, 16), lambda i: (0, i, 0)),
        ],
        out_shape=[
            jax.ShapeDtypeStruct((2, NP, 32), jnp.float32),
            jax.ShapeDtypeStruct((2, NP, 32), jnp.float32),
            jax.ShapeDtypeStruct((NP, 16), jnp.float32),
            jax.ShapeDtypeStruct((K, NP, 16), jnp.float32),
        ],
    )(temp2, jnp.asarray(_CW), jnp.asarray(_BMAT), xpad, W1, b1r, W2, b2r,
      degp)


G = 16
NG = NCHUNK // G
NQ = STRIPE // CH
RING = 4
AHEAD = RING // 2


def _prop_body(w0t, acc0t, d2x, adx, srcp, dstp, accout,
               w_sh, s_sh, acc_v, srcb, dstb, sbuf2, d2xb2, advb2,
               gbuf, zbuf, gsems, ssems, isems, rsems, wsem, zsem):
    cid = lax.axis_index("c")
    sid = lax.axis_index("s")
    nbase = sid * STRIPE
    nsl = pl.ds(nbase, STRIPE)

    def _drain16(sem):
        pltpu.make_async_copy(w0t.at[0].at[pl.ds(0, CH)], gbuf.at[0],
                              sem).wait()

    def _drain8(sem):
        pltpu.make_async_copy(d2x.at[pl.ds(0, CH)], d2xb2.at[0], sem).wait()

    def _drain8i(sem):
        pltpu.make_async_copy(srcp.at[0, pl.ds(0, G)], srcb.at[0], sem).wait()

    pltpu.sync_copy(w0t.at[cid, nsl], w_sh.at[nsl])
    pltpu.sync_copy(acc0t.at[cid, nsl], acc_v)

    def _zfill(r, _):
        zbuf[r, pl.ds(0, 16)] = jnp.zeros((16,), jnp.float32)
        zbuf[r, pl.ds(16, 16)] = jnp.zeros((16,), jnp.float32)
        return 0

    lax.fori_loop(0, CH, _zfill, 0)
    for q in range(NQ):
        pltpu.sync_copy(zbuf, s_sh.at[pl.ds(nbase + q * CH, CH)])
    plsc.subcore_barrier()

    def _gpair(p, _):
        for gg in range(2):
            g = 2 * p + gg
            _group_body(g, gg, 1 - gg)
        return 0

    def _group_body(g, slot, nslot):
        for b in range(G):
            c = g * G + b
            rb = b % RING

            @pl.when(c >= AHEAD)
            def _():
                _drain16(ssems.at[(rb + AHEAD) % RING])

            if b == 4:
                @pl.when(g + 1 < NG)
                def _():
                    pltpu.async_copy(srcp.at[sid, pl.ds((g + 1) * G, G)],
                                     srcb.at[nslot], isems.at[0])
                    pltpu.async_copy(dstp.at[sid, pl.ds((g + 1) * G, G)],
                                     dstb.at[nslot], isems.at[1])

            if b == G - AHEAD:
                @pl.when(g + 1 < NG)
                def _():
                    _drain8i(isems.at[0])
                    _drain8i(isems.at[1])

            if b < G - AHEAD:
                srow = srcb.at[slot, b + AHEAD]
            else:
                srow = srcb.at[nslot, b - (G - AHEAD)]

            @pl.when(c + AHEAD < NCHUNK)
            def _():
                pltpu.async_copy(w_sh.at[srow], gbuf.at[(rb + AHEAD) % RING],
                                 gsems.at[(rb + AHEAD) % RING])

            _drain16(gsems.at[rb])
            pltpu.async_copy(gbuf.at[rb], s_sh.at[dstb.at[slot, b]],
                             ssems.at[rb], add=True)

    def _rload(j, q, slot):
        qsl = pl.ds(nbase + q * CH, CH)
        return [
            pltpu.async_copy(d2x.at[qsl], d2xb2.at[slot], rsems.at[slot]),
            pltpu.async_copy(adx.at[j].at[qsl], advb2.at[slot],
                             rsems.at[slot]),
        ]

    def _rescale(j):
        descs = _rload(j, 0, 0)
        for q in range(NQ):
            slot = q % 2
            nxt = _rload(j, q + 1, 1 - slot) if q + 1 < NQ else []
            pltpu.sync_copy(s_sh.at[pl.ds(nbase + q * CH, CH)],
                            sbuf2.at[slot])
            for d in descs:
                d.wait()
            descs = nxt

            def _row(r, _):
                s0 = sbuf2[slot, r, pl.ds(0, 16)]
                s1 = sbuf2[slot, r, pl.ds(16, 16)]
                ad = advb2[slot, r, :]
                d2 = d2xb2[slot, r, :]
                ar = q * CH + r
                acc_v[ar, pl.ds(0, 16)] = acc_v[ar, pl.ds(0, 16)] + ad * s0
                acc_v[ar, pl.ds(16, 16)] = acc_v[ar, pl.ds(16, 16)] + ad * s1
                sbuf2[slot, r, pl.ds(0, 16)] = d2 * s0
                sbuf2[slot, r, pl.ds(16, 16)] = d2 * s1
                return 0

            lax.fori_loop(0, CH, _row, 0)
            qsl = pl.ds(nbase + q * CH, CH)
            pltpu.sync_copy(sbuf2.at[slot], w_sh.at[qsl])
            pltpu.sync_copy(zbuf, s_sh.at[qsl])

    def _step(j, _):
        pltpu.async_copy(srcp.at[sid, pl.ds(0, G)], srcb.at[0], isems.at[0])
        pltpu.async_copy(dstp.at[sid, pl.ds(0, G)], dstb.at[0], isems.at[1])
        _drain8i(isems.at[0])
        _drain8i(isems.at[1])
        for b in range(AHEAD):
            pltpu.async_copy(w_sh.at[srcb.at[0, b]], gbuf.at[b], gsems.at[b])
        lax.fori_loop(0, NG // 2, _gpair, 0)
        for rb in range(RING - AHEAD, RING):
            _drain16(ssems.at[rb])
        plsc.subcore_barrier()
        _rescale(j)
        plsc.subcore_barrier()
        return 0

    lax.fori_loop(0, K, _step, 0)
    pltpu.sync_copy(acc_v, accout.at[cid, nsl])


_prop_call = functools.partial(
    pl.kernel,
    out_type=jax.ShapeDtypeStruct((2, NP, 32), jnp.float32),
    mesh=_MESH,
    compiler_params=_SC_PARAMS,
    scratch_types=[
        pltpu.VMEM_SHARED((NP, 32), jnp.float32),
        pltpu.VMEM_SHARED((NP, 32), jnp.float32),
        pltpu.VMEM((STRIPE, 32), jnp.float32),
        pltpu.VMEM((2, G, CH), jnp.int32),
        pltpu.VMEM((2, G, CH), jnp.int32),
        pltpu.VMEM((2, CH, 32), jnp.float32),
        pltpu.VMEM((2, CH, 16), jnp.float32),
        pltpu.VMEM((2, CH, 16), jnp.float32),
        pltpu.VMEM((RING, CH, 32), jnp.float32),
        pltpu.VMEM((CH, 32), jnp.float32),
        pltpu.SemaphoreType.DMA((RING,)),
        pltpu.SemaphoreType.DMA((RING,)),
        pltpu.SemaphoreType.DMA((2,)),
        pltpu.SemaphoreType.DMA((2,)),
        pltpu.SemaphoreType.DMA,
        pltpu.SemaphoreType.DMA,
    ],
)(_prop_body)


def kernel(x, edge_index, epoch, W1, b1, W2, b2, temp):
    src = edge_index[0]
    dst = edge_index[1]
    pad = 2 * NT * NCHUNK_D * CH - E
    srcd = jnp.concatenate(
        [src, jnp.full((pad,), DUMMY, jnp.int32)]).reshape(2, NT, NCHUNK_D, CH)
    degp = _deg_call(srcd)

    temp2 = jnp.pad(temp, (0, 16 - (K + 1))).reshape(1, 16)
    xpad = jnp.pad(x, ((0, NP - N), (0, 0)))
    w0t, acc0t, d2x, adx = _tc_call(
        temp2, xpad, W1, b1.reshape(1, HID), W2, b2.reshape(1, HID), degp)

    padp = NT * NCHUNK * CH - E
    srcp = jnp.concatenate(
        [src, jnp.zeros((padp,), jnp.int32)]).reshape(NT, NCHUNK, CH)
    dstp = jnp.concatenate(
        [dst, jnp.full((padp,), DUMMY, jnp.int32)]).reshape(NT, NCHUNK, CH)

    accout = _prop_call(w0t, acc0t, d2x, adx, srcp, dstp)
    return accout.transpose(1, 0, 2).reshape(NP, HID)[:N]


# speedup vs baseline: 1.1424x; 1.0658x over previous
import functools
import math

import jax
import jax.numpy as jnp
import numpy as np
from jax import lax
from jax.experimental import pallas as pl
from jax.experimental.pallas import tpu as pltpu
from jax.experimental.pallas import tpu_sc as plsc

N = 10000
E = 320000
D = 128
HID = 64
K = 10

NT = 16
NP = 10240
STRIPE = NP // NT
CH = 128
NCHUNK = 160
NCHUNK_D = 79
DUMMY = N
BLK = 512

_B = np.zeros((16, 16), np.float64)
for _m in range(K + 1):
    _p = np.array([1.0])
    for _ in range(_m):
        _p = np.convolve(_p, [1.0, -1.0])
    for _ in range(K - _m):
        _p = np.convolve(_p, [1.0, 1.0])
    _B[_m, : len(_p)] = _p
_BMAT = np.asarray(_B, np.float32)
_CWn = np.zeros((1, 16), np.float64)
_CWn[0, : K + 1] = [math.comb(K, m) / 2.0 ** K for m in range(K + 1)]
_CW = np.asarray(_CWn, np.float32)

_MESH = plsc.VectorSubcoreMesh(core_axis_name="c", subcore_axis_name="s")
_SC_PARAMS = pltpu.CompilerParams(use_tc_tiling_on_sc=False)


def _deg_body(srcd, degp, sdeg_sh, idx_v, ones_v, zero_v):
    cid = lax.axis_index("c")
    sid = lax.axis_index("s")
    nbase = sid * STRIPE
    nsl = pl.ds(nbase, STRIPE)

    def _fill(r, _):
        ones_v[r, :] = jnp.full((16,), 1.0, jnp.float32)
        zero_v[r, :] = jnp.zeros((16,), jnp.float32)
        return 0

    lax.fori_loop(0, CH, _fill, 0)
    for q in range(STRIPE // CH):
        pltpu.sync_copy(zero_v, sdeg_sh.at[pl.ds(nbase + q * CH, CH)])
    pltpu.sync_copy(srcd.at[cid, sid], idx_v)
    plsc.subcore_barrier()

    def _chunk(i, _):
        pltpu.sync_copy(ones_v, sdeg_sh.at[idx_v.at[i]], add=True)
        return 0

    lax.fori_loop(0, NCHUNK_D, _chunk, 0)
    plsc.subcore_barrier()
    pltpu.sync_copy(sdeg_sh.at[nsl], degp.at[cid, nsl])


_deg_call = functools.partial(
    pl.kernel,
    out_type=jax.ShapeDtypeStruct((2, NP, 16), jnp.float32),
    mesh=_MESH,
    compiler_params=_SC_PARAMS,
    scratch_types=[
        pltpu.VMEM_SHARED((NP, 16), jnp.float32),
        pltpu.VMEM((NCHUNK_D, CH), jnp.int32),
        pltpu.VMEM((CH, 16), jnp.float32),
        pltpu.VMEM((CH, 16), jnp.float32),
    ],
)(_deg_body)


def _tc_body(temp_ref, cw_ref, bmat_ref, x_ref, w1_ref, b1_ref, w2_ref,
             b2_ref, degp_ref, w0_ref, acc0_ref, d2x_ref, adx_ref):
    h1 = jnp.maximum(x_ref[...] @ w1_ref[...] + b1_ref[...], 0.0)
    h = h1 @ w2_ref[...] + b2_ref[...]
    deg = degp_ref[0, :, 0:1] + degp_ref[1, :, 0:1]
    dinv = jnp.where(deg > 0, lax.rsqrt(deg), 0.0)
    tvec = jnp.maximum(temp_ref[...], 0.0)
    avec = (tvec * cw_ref[...]) @ bmat_ref[...]
    hw = h * dinv
    ha = h * avec[0:1, 0:1]
    w0_ref[...] = jnp.stack([hw[:, :32], hw[:, 32:]], axis=0)
    acc0_ref[...] = jnp.stack([ha[:, :32], ha[:, 32:]], axis=0)
    d2x_ref[...] = jnp.broadcast_to(dinv, (BLK, 16))
    adx_ref[...] = jnp.broadcast_to(avec.T, (16, 16))


def _tc_call(temp2, xpad, W1, b1r, W2, b2r, degp):
    full = lambda s: pl.BlockSpec(s, lambda i: (0,) * len(s))
    return pl.pallas_call(
        _tc_body,
        grid=(NP // BLK,),
        in_specs=[
            full((1, 16)),
            full((1, 16)),
            full((16, 16)),
            pl.BlockSpec((BLK, D), lambda i: (i, 0)),
            full((D, HID)),
            full((1, HID)),
            full((HID, HID)),
            full((1, HID)),
            pl.BlockSpec((2, BLK, 16), lambda i: (0, i, 0)),
        ],
        out_specs=[
            pl.BlockSpec((2, BLK, 32), lambda i: (0, i, 0)),
            pl.BlockSpec((2, BLK, 32), lambda i: (0, i, 0)),
            pl.BlockSpec((BLK, 16), lambda i: (i, 0)),
            pl.BlockSpec((16, 16), lambda i: (0, 0)),
        ],
        out_shape=[
            jax.ShapeDtypeStruct((2, NP, 32), jnp.float32),
            jax.ShapeDtypeStruct((2, NP, 32), jnp.float32),
            jax.ShapeDtypeStruct((NP, 16), jnp.float32),
            jax.ShapeDtypeStruct((16, 16), jnp.float32),
        ],
    )(temp2, jnp.asarray(_CW), jnp.asarray(_BMAT), xpad, W1, b1r, W2, b2r,
      degp)


G = 16
NG = NCHUNK // G
NQ = STRIPE // CH
RING = 4
AHEAD = RING // 2


def _prop_body(w0t, acc0t, dinvb, a16, srcp, dstp, accout,
               w_sh, s_sh, acc_v, srcb, dstb, sbuf, dinvb_v, a16_v,
               gbuf, zbuf, gsems, ssems, isems):
    cid = lax.axis_index("c")
    sid = lax.axis_index("s")
    nbase = sid * STRIPE
    nsl = pl.ds(nbase, STRIPE)

    def _drain16(sem):
        pltpu.make_async_copy(w0t.at[0].at[pl.ds(0, CH)], gbuf.at[0],
                              sem).wait()

    def _drain8i(sem):
        pltpu.make_async_copy(srcp.at[0, pl.ds(0, G)], srcb.at[0], sem).wait()

    pltpu.sync_copy(w0t.at[cid, nsl], w_sh.at[nsl])
    pltpu.sync_copy(acc0t.at[cid, nsl], acc_v)
    pltpu.sync_copy(dinvb.at[nsl], dinvb_v)
    pltpu.sync_copy(a16, a16_v)

    def _zfill(r, _):
        zbuf[r, pl.ds(0, 16)] = jnp.zeros((16,), jnp.float32)
        zbuf[r, pl.ds(16, 16)] = jnp.zeros((16,), jnp.float32)
        return 0

    lax.fori_loop(0, CH, _zfill, 0)
    for q in range(NQ):
        pltpu.sync_copy(zbuf, s_sh.at[pl.ds(nbase + q * CH, CH)])
    plsc.subcore_barrier()

    def _gpair(p, _):
        for gg in range(2):
            g = 2 * p + gg
            _group_body(g, gg, 1 - gg)
        return 0

    def _group_body(g, slot, nslot):
        for b in range(G):
            c = g * G + b
            rb = b % RING

            @pl.when(c >= AHEAD)
            def _():
                _drain16(ssems.at[(rb + AHEAD) % RING])

            if b == 4:
                @pl.when(g + 1 < NG)
                def _():
                    pltpu.async_copy(srcp.at[sid, pl.ds((g + 1) * G, G)],
                                     srcb.at[nslot], isems.at[0])
                    pltpu.async_copy(dstp.at[sid, pl.ds((g + 1) * G, G)],
                                     dstb.at[nslot], isems.at[1])

            if b == G - AHEAD:
                @pl.when(g + 1 < NG)
                def _():
                    _drain8i(isems.at[0])
                    _drain8i(isems.at[1])

            if b < G - AHEAD:
                srow = srcb.at[slot, b + AHEAD]
            else:
                srow = srcb.at[nslot, b - (G - AHEAD)]

            @pl.when(c + AHEAD < NCHUNK)
            def _():
                pltpu.async_copy(w_sh.at[srow], gbuf.at[(rb + AHEAD) % RING],
                                 gsems.at[(rb + AHEAD) % RING])

            _drain16(gsems.at[rb])
            pltpu.async_copy(gbuf.at[rb], s_sh.at[dstb.at[slot, b]],
                             ssems.at[rb], add=True)

    def _rescale(j):
        aj = a16_v[j + 1, :]
        for q in range(NQ):
            qsl = pl.ds(nbase + q * CH, CH)
            pltpu.sync_copy(s_sh.at[qsl], sbuf)

            def _row(r, _):
                ar = q * CH + r
                s0 = sbuf[r, pl.ds(0, 16)]
                s1 = sbuf[r, pl.ds(16, 16)]
                d = dinvb_v[ar, :]
                ad = aj * d
                d2 = d * d
                acc_v[ar, pl.ds(0, 16)] = acc_v[ar, pl.ds(0, 16)] + ad * s0
                acc_v[ar, pl.ds(16, 16)] = acc_v[ar, pl.ds(16, 16)] + ad * s1
                sbuf[r, pl.ds(0, 16)] = d2 * s0
                sbuf[r, pl.ds(16, 16)] = d2 * s1
                return 0

            lax.fori_loop(0, CH, _row, 0)
            pltpu.sync_copy(sbuf, w_sh.at[qsl])
            pltpu.sync_copy(zbuf, s_sh.at[qsl])

    def _step(j, _):
        pltpu.async_copy(srcp.at[sid, pl.ds(0, G)], srcb.at[0], isems.at[0])
        pltpu.async_copy(dstp.at[sid, pl.ds(0, G)], dstb.at[0], isems.at[1])
        _drain8i(isems.at[0])
        _drain8i(isems.at[1])
        for b in range(AHEAD):
            pltpu.async_copy(w_sh.at[srcb.at[0, b]], gbuf.at[b], gsems.at[b])
        lax.fori_loop(0, NG // 2, _gpair, 0)
        for rb in range(RING - AHEAD, RING):
            _drain16(ssems.at[rb])
        plsc.subcore_barrier()
        _rescale(j)
        plsc.subcore_barrier()
        return 0

    lax.fori_loop(0, K, _step, 0)
    pltpu.sync_copy(acc_v, accout.at[cid, nsl])


_prop_call = functools.partial(
    pl.kernel,
    out_type=jax.ShapeDtypeStruct((2, NP, 32), jnp.float32),
    mesh=_MESH,
    compiler_params=_SC_PARAMS,
    scratch_types=[
        pltpu.VMEM_SHARED((NP, 32), jnp.float32),
        pltpu.VMEM_SHARED((NP, 32), jnp.float32),
        pltpu.VMEM((STRIPE, 32), jnp.float32),
        pltpu.VMEM((2, G, CH), jnp.int32),
        pltpu.VMEM((2, G, CH), jnp.int32),
        pltpu.VMEM((CH, 32), jnp.float32),
        pltpu.VMEM((STRIPE, 16), jnp.float32),
        pltpu.VMEM((16, 16), jnp.float32),
        pltpu.VMEM((RING, CH, 32), jnp.float32),
        pltpu.VMEM((CH, 32), jnp.float32),
        pltpu.SemaphoreType.DMA((RING,)),
        pltpu.SemaphoreType.DMA((RING,)),
        pltpu.SemaphoreType.DMA((2,)),
    ],
)(_prop_body)


def kernel(x, edge_index, epoch, W1, b1, W2, b2, temp):
    src = edge_index[0]
    dst = edge_index[1]
    pad = 2 * NT * NCHUNK_D * CH - E
    srcd = jnp.concatenate(
        [src, jnp.full((pad,), DUMMY, jnp.int32)]).reshape(2, NT, NCHUNK_D, CH)
    degp = _deg_call(srcd)

    temp2 = jnp.pad(temp, (0, 16 - (K + 1))).reshape(1, 16)
    xpad = jnp.pad(x, ((0, NP - N), (0, 0)))
    w0t, acc0t, d2x, adx = _tc_call(
        temp2, xpad, W1, b1.reshape(1, HID), W2, b2.reshape(1, HID), degp)

    padp = NT * NCHUNK * CH - E
    srcp = jnp.concatenate(
        [src, jnp.zeros((padp,), jnp.int32)]).reshape(NT, NCHUNK, CH)
    dstp = jnp.concatenate(
        [dst, jnp.full((padp,), DUMMY, jnp.int32)]).reshape(NT, NCHUNK, CH)

    accout = _prop_call(w0t, acc0t, d2x, adx, srcp, dstp)
    return accout.transpose(1, 0, 2).reshape(NP, HID)[:N]


# speedup vs baseline: 1.1719x; 1.0258x over previous
import functools
import math

import jax
import jax.numpy as jnp
import numpy as np
from jax import lax
from jax.experimental import pallas as pl
from jax.experimental.pallas import tpu as pltpu
from jax.experimental.pallas import tpu_sc as plsc

N = 10000
E = 320000
D = 128
HID = 64
K = 10

NT = 16
NP = 10240
STRIPE = NP // NT
CH = 128
NCHUNK = 160
NCHUNK_D = 79
DUMMY = N
BLK = 1024

_B = np.zeros((16, 16), np.float64)
for _m in range(K + 1):
    _p = np.array([1.0])
    for _ in range(_m):
        _p = np.convolve(_p, [1.0, -1.0])
    for _ in range(K - _m):
        _p = np.convolve(_p, [1.0, 1.0])
    _B[_m, : len(_p)] = _p
_BMAT = np.asarray(_B, np.float32)
_CWn = np.zeros((1, 16), np.float64)
_CWn[0, : K + 1] = [math.comb(K, m) / 2.0 ** K for m in range(K + 1)]
_CW = np.asarray(_CWn, np.float32)

_MESH = plsc.VectorSubcoreMesh(core_axis_name="c", subcore_axis_name="s")
_SC_PARAMS = pltpu.CompilerParams(use_tc_tiling_on_sc=False)


def _deg_body(srcd, degp, sdeg_sh, idx_v, ones_v, zero_v):
    cid = lax.axis_index("c")
    sid = lax.axis_index("s")
    nbase = sid * STRIPE
    nsl = pl.ds(nbase, STRIPE)

    def _fill(r, _):
        ones_v[r, :] = jnp.full((16,), 1.0, jnp.float32)
        zero_v[r, :] = jnp.zeros((16,), jnp.float32)
        return 0

    lax.fori_loop(0, CH, _fill, 0)
    for q in range(STRIPE // CH):
        pltpu.sync_copy(zero_v, sdeg_sh.at[pl.ds(nbase + q * CH, CH)])
    pltpu.sync_copy(srcd.at[cid, sid], idx_v)
    plsc.subcore_barrier()

    def _chunk(i, _):
        pltpu.sync_copy(ones_v, sdeg_sh.at[idx_v.at[i]], add=True)
        return 0

    lax.fori_loop(0, NCHUNK_D, _chunk, 0)
    plsc.subcore_barrier()
    pltpu.sync_copy(sdeg_sh.at[nsl], degp.at[cid, nsl])


_deg_call = functools.partial(
    pl.kernel,
    out_type=jax.ShapeDtypeStruct((2, NP, 16), jnp.float32),
    mesh=_MESH,
    compiler_params=_SC_PARAMS,
    scratch_types=[
        pltpu.VMEM_SHARED((NP, 16), jnp.float32),
        pltpu.VMEM((NCHUNK_D, CH), jnp.int32),
        pltpu.VMEM((CH, 16), jnp.float32),
        pltpu.VMEM((CH, 16), jnp.float32),
    ],
)(_deg_body)


def _tc_body(temp_ref, cw_ref, bmat_ref, x_ref, w1_ref, b1_ref, w2_ref,
             b2_ref, degp_ref, w0_ref, acc0_ref, d2x_ref, adx_ref):
    h1 = jnp.maximum(x_ref[...] @ w1_ref[...] + b1_ref[...], 0.0)
    h = h1 @ w2_ref[...] + b2_ref[...]
    deg = degp_ref[0, :, 0:1] + degp_ref[1, :, 0:1]
    dinv = jnp.where(deg > 0, lax.rsqrt(deg), 0.0)
    tvec = jnp.maximum(temp_ref[...], 0.0)
    avec = (tvec * cw_ref[...]) @ bmat_ref[...]
    hw = h * dinv
    ha = h * avec[0:1, 0:1]
    w0_ref[...] = jnp.stack([hw[:, :32], hw[:, 32:]], axis=0)
    acc0_ref[...] = jnp.stack([ha[:, :32], ha[:, 32:]], axis=0)
    d2x_ref[...] = jnp.broadcast_to(dinv, (BLK, 16))
    adx_ref[...] = jnp.broadcast_to(avec.T, (16, 16))


def _tc_call(temp2, xpad, W1, b1r, W2, b2r, degp):
    full = lambda s: pl.BlockSpec(s, lambda i: (0,) * len(s))
    return pl.pallas_call(
        _tc_body,
        grid=(NP // BLK,),
        in_specs=[
            full((1, 16)),
            full((1, 16)),
            full((16, 16)),
            pl.BlockSpec((BLK, D), lambda i: (i, 0)),
            full((D, HID)),
            full((1, HID)),
            full((HID, HID)),
            full((1, HID)),
            pl.BlockSpec((2, BLK, 16), lambda i: (0, i, 0)),
        ],
        out_specs=[
            pl.BlockSpec((2, BLK, 32), lambda i: (0, i, 0)),
            pl.BlockSpec((2, BLK, 32), lambda i: (0, i, 0)),
            pl.BlockSpec((BLK, 16), lambda i: (i, 0)),
            pl.BlockSpec((16, 16), lambda i: (0, 0)),
        ],
        out_shape=[
            jax.ShapeDtypeStruct((2, NP, 32), jnp.float32),
            jax.ShapeDtypeStruct((2, NP, 32), jnp.float32),
            jax.ShapeDtypeStruct((NP, 16), jnp.float32),
            jax.ShapeDtypeStruct((16, 16), jnp.float32),
        ],
    )(temp2, jnp.asarray(_CW), jnp.asarray(_BMAT), xpad, W1, b1r, W2, b2r,
      degp)


G = 16
NG = NCHUNK // G
NQ = STRIPE // CH
RING = 4
AHEAD = RING // 2


def _prop_body(w0t, acc0t, dinvb, a16, srcp, dstp, accout,
               w_sh, s_sh, acc_v, srcb, dstb, sbuf, dinvb_v, a16_v,
               gbuf, zbuf, gsems, ssems, isems):
    cid = lax.axis_index("c")
    sid = lax.axis_index("s")
    nbase = sid * STRIPE
    nsl = pl.ds(nbase, STRIPE)

    def _drain16(sem):
        pltpu.make_async_copy(w0t.at[0].at[pl.ds(0, CH)], gbuf.at[0],
                              sem).wait()

    def _drain8i(sem):
        pltpu.make_async_copy(srcp.at[0, pl.ds(0, G)], srcb.at[0], sem).wait()

    pltpu.sync_copy(w0t.at[cid, nsl], w_sh.at[nsl])
    pltpu.sync_copy(acc0t.at[cid, nsl], acc_v)
    pltpu.sync_copy(dinvb.at[nsl], dinvb_v)
    pltpu.sync_copy(a16, a16_v)

    def _zfill(r, _):
        zbuf[r, pl.ds(0, 16)] = jnp.zeros((16,), jnp.float32)
        zbuf[r, pl.ds(16, 16)] = jnp.zeros((16,), jnp.float32)
        return 0

    lax.fori_loop(0, CH, _zfill, 0)
    for q in range(NQ):
        pltpu.sync_copy(zbuf, s_sh.at[pl.ds(nbase + q * CH, CH)])
    plsc.subcore_barrier()

    def _gpair(p, _):
        for gg in range(2):
            g = 2 * p + gg
            _group_body(g, gg, 1 - gg)
        return 0

    def _group_body(g, slot, nslot):
        for b in range(G):
            c = g * G + b
            rb = b % RING

            @pl.when(c >= AHEAD)
            def _():
                _drain16(ssems.at[(rb + AHEAD) % RING])

            if b == 4:
                @pl.when(g + 1 < NG)
                def _():
                    pltpu.async_copy(srcp.at[sid, pl.ds((g + 1) * G, G)],
                                     srcb.at[nslot], isems.at[0])
                    pltpu.async_copy(dstp.at[sid, pl.ds((g + 1) * G, G)],
                                     dstb.at[nslot], isems.at[1])

            if b == G - AHEAD:
                @pl.when(g + 1 < NG)
                def _():
                    _drain8i(isems.at[0])
                    _drain8i(isems.at[1])

            if b < G - AHEAD:
                srow = srcb.at[slot, b + AHEAD]
            else:
                srow = srcb.at[nslot, b - (G - AHEAD)]

            @pl.when(c + AHEAD < NCHUNK)
            def _():
                pltpu.async_copy(w_sh.at[srow], gbuf.at[(rb + AHEAD) % RING],
                                 gsems.at[(rb + AHEAD) % RING])

            _drain16(gsems.at[rb])
            pltpu.async_copy(gbuf.at[rb], s_sh.at[dstb.at[slot, b]],
                             ssems.at[rb], add=True)

    def _rescale(j):
        aj = a16_v[j + 1, :]
        for q in range(NQ):
            qsl = pl.ds(nbase + q * CH, CH)
            pltpu.sync_copy(s_sh.at[qsl], sbuf)

            def _row(r, _):
                ar = q * CH + r
                s0 = sbuf[r, pl.ds(0, 16)]
                s1 = sbuf[r, pl.ds(16, 16)]
                d = dinvb_v[ar, :]
                ad = aj * d
                d2 = d * d
                acc_v[ar, pl.ds(0, 16)] = acc_v[ar, pl.ds(0, 16)] + ad * s0
                acc_v[ar, pl.ds(16, 16)] = acc_v[ar, pl.ds(16, 16)] + ad * s1
                sbuf[r, pl.ds(0, 16)] = d2 * s0
                sbuf[r, pl.ds(16, 16)] = d2 * s1
                return 0

            lax.fori_loop(0, CH, _row, 0)
            pltpu.sync_copy(sbuf, w_sh.at[qsl])
            pltpu.sync_copy(zbuf, s_sh.at[qsl])

    def _step(j, _):
        pltpu.async_copy(srcp.at[sid, pl.ds(0, G)], srcb.at[0], isems.at[0])
        pltpu.async_copy(dstp.at[sid, pl.ds(0, G)], dstb.at[0], isems.at[1])
        _drain8i(isems.at[0])
        _drain8i(isems.at[1])
        for b in range(AHEAD):
            pltpu.async_copy(w_sh.at[srcb.at[0, b]], gbuf.at[b], gsems.at[b])
        lax.fori_loop(0, NG // 2, _gpair, 0)
        for rb in range(RING - AHEAD, RING):
            _drain16(ssems.at[rb])
        plsc.subcore_barrier()
        _rescale(j)
        plsc.subcore_barrier()
        return 0

    lax.fori_loop(0, K, _step, 0)
    pltpu.sync_copy(acc_v, accout.at[nsl, pl.ds(cid * 32, 32)])


_prop_call = functools.partial(
    pl.kernel,
    out_type=jax.ShapeDtypeStruct((NP, HID), jnp.float32),
    mesh=_MESH,
    compiler_params=_SC_PARAMS,
    scratch_types=[
        pltpu.VMEM_SHARED((NP, 32), jnp.float32),
        pltpu.VMEM_SHARED((NP, 32), jnp.float32),
        pltpu.VMEM((STRIPE, 32), jnp.float32),
        pltpu.VMEM((2, G, CH), jnp.int32),
        pltpu.VMEM((2, G, CH), jnp.int32),
        pltpu.VMEM((CH, 32), jnp.float32),
        pltpu.VMEM((STRIPE, 16), jnp.float32),
        pltpu.VMEM((16, 16), jnp.float32),
        pltpu.VMEM((RING, CH, 32), jnp.float32),
        pltpu.VMEM((CH, 32), jnp.float32),
        pltpu.SemaphoreType.DMA((RING,)),
        pltpu.SemaphoreType.DMA((RING,)),
        pltpu.SemaphoreType.DMA((2,)),
    ],
)(_prop_body)


def kernel(x, edge_index, epoch, W1, b1, W2, b2, temp):
    src = edge_index[0]
    dst = edge_index[1]
    pad = 2 * NT * NCHUNK_D * CH - E
    srcd = jnp.concatenate(
        [src, jnp.full((pad,), DUMMY, jnp.int32)]).reshape(2, NT, NCHUNK_D, CH)
    degp = _deg_call(srcd)

    temp2 = jnp.pad(temp, (0, 16 - (K + 1))).reshape(1, 16)
    xpad = jnp.pad(x, ((0, NP - N), (0, 0)))
    w0t, acc0t, d2x, adx = _tc_call(
        temp2, xpad, W1, b1.reshape(1, HID), W2, b2.reshape(1, HID), degp)

    padp = NT * NCHUNK * CH - E
    srcp = jnp.concatenate(
        [src, jnp.zeros((padp,), jnp.int32)]).reshape(NT, NCHUNK, CH)
    dstp = jnp.concatenate(
        [dst, jnp.full((padp,), DUMMY, jnp.int32)]).reshape(NT, NCHUNK, CH)

    accout = _prop_call(w0t, acc0t, d2x, adx, srcp, dstp)
    return accout[:N]
